# baseline (speedup 1.0000x reference)
import functools

import jax
import jax.numpy as jnp
import numpy as np
from jax import lax
from jax.experimental import pallas as pl
from jax.experimental.pallas import tpu as pltpu
from jax.experimental.pallas import tpu_sc as plsc

_SIZES = [1, 2, 4, 9]
_F = 26
_V = 100000
_E = 16
_B = 4096
_K = len(_SIZES)
_NC, _NS = 2, 16
_NW = _NC * _NS
_N = _B * _F
_RPW = _N // _NW


def _nas_masks():
    rows = []
    for i, m in enumerate(_SIZES):
        before = sum(_SIZES[:i])
        rows.append([0.0] * before + [1.0] * m + [0.0] * (_E - m - before))
    return np.array(rows, dtype=np.float32)


def _blockdiag_masks():
    m = _nas_masks()
    out = np.zeros((_F * _K, _F * _E), dtype=np.float32)
    for f in range(_F):
        out[f * _K:(f + 1) * _K, f * _E:(f + 1) * _E] = m
    return out


_MASKS_BD = _blockdiag_masks()

@functools.cache
def _make_sc_gather():
    mesh = plsc.VectorSubcoreMesh(core_axis_name="c", subcore_axis_name="s")

    @functools.partial(
        pl.kernel,
        mesh=mesh,
        compiler_params=pltpu.CompilerParams(use_tc_tiling_on_sc=False),
        out_type=(
            jax.ShapeDtypeStruct((_N, _E), jnp.float32),
            jax.ShapeDtypeStruct((_N,), jnp.float32),
        ),
        scratch_types=[
            pltpu.VMEM((_RPW,), jnp.int32),
            pltpu.VMEM((_RPW, _E), jnp.float32),
            pltpu.VMEM((_RPW,), jnp.float32),
            pltpu.SemaphoreType.DMA,
            pltpu.SemaphoreType.DMA,
        ],
    )
    def _sc_gather(emb_hbm, bias_hbm, idx_hbm, out_emb, out_bias,
                   idx_v, emb_v, bias_v, sem_e, sem_b):
        wid = lax.axis_index("s") * _NC + lax.axis_index("c")
        base = wid * _RPW
        pltpu.sync_copy(idx_hbm.at[pl.ds(base, _RPW)], idx_v)
        ce = pltpu.async_copy(emb_hbm.at[idx_v], emb_v, sem_e)
        cb = pltpu.async_copy(bias_hbm.at[idx_v], bias_v, sem_b)
        ce.wait()
        cb.wait()
        pltpu.sync_copy(emb_v, out_emb.at[pl.ds(base, _RPW)])
        pltpu.sync_copy(bias_v, out_bias.at[pl.ds(base, _RPW)])

    return _sc_gather


def _tc_body(x_ref, p_ref, m_ref, bias_ref, w1_ref, b1_ref, w2_ref, b2_ref,
             w3_ref, b3_ref, out_ref):
    x = x_ref[...]
    mean = jnp.mean(x, axis=0, keepdims=True)
    xc = x - mean
    var = jnp.mean(xc * xc, axis=0, keepdims=True)
    cm = jnp.dot(p_ref[...], m_ref[...],
                 preferred_element_type=jnp.float32)
    scale = cm * lax.rsqrt(var + 1e-3)
    xn = xc * scale
    h = jnp.dot(xn, w1_ref[...], precision=lax.Precision.HIGHEST,
                preferred_element_type=jnp.float32) + b1_ref[...]
    h = jnp.maximum(h, 0.0)
    h = jnp.dot(h, w2_ref[...], precision=lax.Precision.HIGHEST,
                preferred_element_type=jnp.float32) + b2_ref[...]
    h = jnp.maximum(h, 0.0)
    y = jnp.dot(h, w3_ref[...], precision=lax.Precision.HIGHEST,
                preferred_element_type=jnp.float32) + b3_ref[...]
    bsum = jnp.sum(bias_ref[...], axis=1, keepdims=True)
    out_ref[...] = jax.nn.sigmoid(y + bsum)


_tc_forward = pl.pallas_call(
    _tc_body,
    out_shape=jax.ShapeDtypeStruct((_B, 1), jnp.float32),
)


def kernel(inputs, emb_tables, bias_tables, nas_logits, W1, b1, W2, b2, W3, b3):
    offs = (jnp.arange(_F, dtype=jnp.int32) * _V)[None, :]
    idx = (inputs.astype(jnp.int32) + offs).reshape(-1)
    emb_flat = emb_tables.reshape(_F * _V, _E)
    bias_flat = bias_tables.reshape(_F * _V)
    embs, bias_vals = _make_sc_gather()(emb_flat, bias_flat, idx)
    x = embs.reshape(_B, _F * _E)
    bias2d = bias_vals.reshape(_B, _F)
    p = jax.nn.softmax(nas_logits, axis=1).reshape(1, _F * _K)
    out = _tc_forward(x, p, jnp.asarray(_MASKS_BD), bias2d,
                      W1, b1.reshape(1, -1), W2, b2.reshape(1, -1),
                      W3, b3.reshape(1, 1))
    return out.reshape(_B)


# speedup vs baseline: 1.0195x; 1.0195x over previous
import functools

import jax
import jax.numpy as jnp
import numpy as np
from jax import lax
from jax.experimental import pallas as pl
from jax.experimental.pallas import tpu as pltpu
from jax.experimental.pallas import tpu_sc as plsc

_SIZES = [1, 2, 4, 9]
_F = 26
_V = 100000
_E = 16
_B = 4096
_K = len(_SIZES)
_NC, _NS = 2, 16
_NW = _NC * _NS
_N = _B * _F
_RPW = _N // _NW


def _nas_masks():
    rows = []
    for i, m in enumerate(_SIZES):
        before = sum(_SIZES[:i])
        rows.append([0.0] * before + [1.0] * m + [0.0] * (_E - m - before))
    return np.array(rows, dtype=np.float32)


def _blockdiag_masks():
    m = _nas_masks()
    out = np.zeros((_F * _K, _F * _E), dtype=np.float32)
    for f in range(_F):
        out[f * _K:(f + 1) * _K, f * _E:(f + 1) * _E] = m
    return out


_MASKS_BD = _blockdiag_masks()

@functools.cache
def _make_sc_gather():
    mesh = plsc.VectorSubcoreMesh(core_axis_name="c", subcore_axis_name="s")

    @functools.partial(
        pl.kernel,
        mesh=mesh,
        compiler_params=pltpu.CompilerParams(use_tc_tiling_on_sc=False),
        out_type=(
            jax.ShapeDtypeStruct((_N, _E), jnp.float32),
            jax.ShapeDtypeStruct((_N, 8), jnp.float32),
        ),
        scratch_types=[
            pltpu.VMEM((_RPW,), jnp.int32),
            pltpu.VMEM((_RPW,), jnp.int32),
            pltpu.VMEM((_RPW, _E), jnp.float32),
            pltpu.VMEM((_RPW, 8), jnp.float32),
            pltpu.SemaphoreType.DMA,
            pltpu.SemaphoreType.DMA,
        ],
    )
    def _sc_gather(emb_hbm, bias8_hbm, idx_hbm, ridx_hbm, out_emb, out_bias8,
                   idx_v, ridx_v, emb_v, b8_v, sem_e, sem_b):
        wid = lax.axis_index("s") * _NC + lax.axis_index("c")
        base = wid * _RPW
        pltpu.sync_copy(idx_hbm.at[pl.ds(base, _RPW)], idx_v)
        pltpu.sync_copy(ridx_hbm.at[pl.ds(base, _RPW)], ridx_v)
        ce = pltpu.async_copy(emb_hbm.at[idx_v], emb_v, sem_e)
        cb = pltpu.async_copy(bias8_hbm.at[ridx_v], b8_v, sem_b)
        ce.wait()
        cb.wait()
        pltpu.sync_copy(emb_v, out_emb.at[pl.ds(base, _RPW)])
        pltpu.sync_copy(b8_v, out_bias8.at[pl.ds(base, _RPW)])

    return _sc_gather


def _tc_body(x_ref, p_ref, m_ref, bias_ref, lane_ref, w1_ref, b1_ref, w2_ref,
             b2_ref, w3_ref, b3_ref, out_ref):
    x = x_ref[...]
    mean = jnp.mean(x, axis=0, keepdims=True)
    xc = x - mean
    var = jnp.mean(xc * xc, axis=0, keepdims=True)
    cm = jnp.dot(p_ref[...], m_ref[...],
                 preferred_element_type=jnp.float32)
    scale = cm * lax.rsqrt(var + 1e-3)
    xn = xc * scale
    h = jnp.dot(xn, w1_ref[...],
                preferred_element_type=jnp.float32) + b1_ref[...]
    h = jnp.maximum(h, 0.0)
    h = jnp.dot(h, w2_ref[...],
                preferred_element_type=jnp.float32) + b2_ref[...]
    h = jnp.maximum(h, 0.0)
    y = jnp.dot(h, w3_ref[...],
                preferred_element_type=jnp.float32) + b3_ref[...]
    b8 = bias_ref[...]
    lanes = lane_ref[...]
    sub = lax.broadcasted_iota(jnp.int32, b8.shape, 1) % 8
    picked = jnp.where(lanes == sub, b8, 0.0)
    bsum = jnp.sum(picked, axis=1, keepdims=True)
    out_ref[...] = jax.nn.sigmoid(y + bsum)


_tc_forward = pl.pallas_call(
    _tc_body,
    out_shape=jax.ShapeDtypeStruct((_B, 1), jnp.float32),
)


def kernel(inputs, emb_tables, bias_tables, nas_logits, W1, b1, W2, b2, W3, b3):
    inp = inputs.astype(jnp.int32)
    offs = (jnp.arange(_F, dtype=jnp.int32) * _V)[None, :]
    idx = (inp + offs).reshape(-1)
    ridx = idx >> 3
    lane_exp = jnp.repeat(inp & 7, 8, axis=1)
    emb_flat = emb_tables.reshape(_F * _V, _E)
    bias8 = bias_tables.reshape(_F * _V // 8, 8)
    embs, bias8g = _make_sc_gather()(emb_flat, bias8, idx, ridx)
    x = embs.reshape(_B, _F * _E)
    bias8b = bias8g.reshape(_B, _F * 8)
    p = jax.nn.softmax(nas_logits, axis=1).reshape(1, _F * _K)
    out = _tc_forward(x, p, jnp.asarray(_MASKS_BD), bias8b, lane_exp,
                      W1, b1.reshape(1, -1), W2, b2.reshape(1, -1),
                      W3, b3.reshape(1, 1))
    return out.reshape(_B)


# speedup vs baseline: 2.1078x; 2.0675x over previous
import functools

import jax
import jax.numpy as jnp
import numpy as np
from jax import lax
from jax.experimental import pallas as pl
from jax.experimental.pallas import tpu as pltpu
from jax.experimental.pallas import tpu_sc as plsc

_SIZES = [1, 2, 4, 9]
_F = 26
_V = 100000
_E = 16
_B = 4096
_K = len(_SIZES)
_NC, _NS = 2, 16
_NW = _NC * _NS
_N = _B * _F
_RPW = _N // _NW
_CHK = 208


def _nas_masks():
    rows = []
    for i, m in enumerate(_SIZES):
        before = sum(_SIZES[:i])
        rows.append([0.0] * before + [1.0] * m + [0.0] * (_E - m - before))
    return np.array(rows, dtype=np.float32)


def _blockdiag_masks():
    m = _nas_masks()
    out = np.zeros((_F * _K, _F * _E), dtype=np.float32)
    for f in range(_F):
        out[f * _K:(f + 1) * _K, f * _E:(f + 1) * _E] = m
    return out


_MASKS_BD = _blockdiag_masks()

@functools.cache
def _make_sc_gather():
    mesh = plsc.VectorSubcoreMesh(core_axis_name="c", subcore_axis_name="s")

    @functools.partial(
        pl.kernel,
        mesh=mesh,
        compiler_params=pltpu.CompilerParams(
            use_tc_tiling_on_sc=False, needs_layout_passes=False),
        out_type=(
            jax.ShapeDtypeStruct((_N, _E), jnp.float32),
            jax.ShapeDtypeStruct((_N, 8), jnp.float32),
        ),
        scratch_types=[
            pltpu.VMEM((_RPW,), jnp.int32),
            pltpu.VMEM((_RPW,), jnp.int32),
            pltpu.VMEM((_RPW,), jnp.int32),
            pltpu.VMEM((_E * _CHK,), jnp.int32),
            pltpu.VMEM((_E * _CHK, 8), jnp.float32),
            pltpu.VMEM((_RPW, _E), jnp.float32),
            pltpu.VMEM((_RPW, 8), jnp.float32),
            pltpu.SemaphoreType.DMA,
            pltpu.SemaphoreType.DMA,
        ],
    )
    def _sc_gather(emb8_hbm, bias8_hbm, ridx0_hbm, lane_hbm, bridx_hbm,
                   out_emb, out_bias8, ridx0_v, lane_v, bridx_v, ge_v, blk_v,
                   emb_v, b8_v, sem_e, sem_b):
        wid = lax.axis_index("s") * _NC + lax.axis_index("c")
        base = wid * _RPW
        pltpu.sync_copy(ridx0_hbm.at[pl.ds(base, _RPW)], ridx0_v)
        pltpu.sync_copy(lane_hbm.at[pl.ds(base, _RPW)], lane_v)
        pltpu.sync_copy(bridx_hbm.at[pl.ds(base, _RPW)], bridx_v)
        pltpu.async_copy(bias8_hbm.at[bridx_v], b8_v, sem_b).wait()
        pltpu.sync_copy(b8_v, out_bias8.at[pl.ds(base, _RPW)])

        def _chunk(c, _):
            c0 = c * _CHK

            def _fill(g, _):
                j0 = g * 16
                rbase = ridx0_v[pl.ds(c0 + j0, 16)]
                for e in range(_E):
                    ge_v[pl.ds(e * _CHK + j0, 16)] = rbase + e * (_V // 8)
                return 0

            lax.fori_loop(0, _CHK // 16, _fill, 0)
            copies = [
                pltpu.async_copy(
                    emb8_hbm.at[ge_v.at[pl.ds(e * _CHK, _CHK)]],
                    blk_v.at[pl.ds(e * _CHK, _CHK)], sem_e)
                for e in range(_E)
            ]
            for cp in copies:
                cp.wait()

            def _extract(g, _):
                j0 = g * 16
                rows = lax.iota(jnp.int32, 16)
                lanes = lane_v[pl.ds(c0 + j0, 16)]
                dst = emb_v.at[pl.ds(c0 + j0, 16)]
                for e in range(_E):
                    vals = plsc.load_gather(
                        blk_v.at[pl.ds(e * _CHK + j0, 16)], [rows, lanes])
                    plsc.store_scatter(
                        dst, [rows, jnp.full((16,), e, jnp.int32)], vals)
                return 0

            lax.fori_loop(0, _CHK // 16, _extract, 0)
            return 0

        lax.fori_loop(0, _RPW // _CHK, _chunk, 0)
        pltpu.sync_copy(emb_v, out_emb.at[pl.ds(base, _RPW)])

    return _sc_gather


def _tc_body(x_ref, p_ref, m_ref, bias_ref, lane_ref, w1_ref, b1_ref, w2_ref,
             b2_ref, w3_ref, b3_ref, out_ref):
    x = x_ref[...]
    mean = jnp.mean(x, axis=0, keepdims=True)
    xc = x - mean
    var = jnp.mean(xc * xc, axis=0, keepdims=True)
    cm = jnp.dot(p_ref[...], m_ref[...],
                 preferred_element_type=jnp.float32)
    scale = cm * lax.rsqrt(var + 1e-3)
    xn = xc * scale
    h = jnp.dot(xn, w1_ref[...],
                preferred_element_type=jnp.float32) + b1_ref[...]
    h = jnp.maximum(h, 0.0)
    h = jnp.dot(h, w2_ref[...],
                preferred_element_type=jnp.float32) + b2_ref[...]
    h = jnp.maximum(h, 0.0)
    y = jnp.dot(h, w3_ref[...],
                preferred_element_type=jnp.float32) + b3_ref[...]
    b8 = bias_ref[...]
    lanes = lane_ref[...]
    sub = lax.broadcasted_iota(jnp.int32, b8.shape, 1) % 8
    picked = jnp.where(lanes == sub, b8, 0.0)
    bsum = jnp.sum(picked, axis=1, keepdims=True)
    out_ref[...] = jax.nn.sigmoid(y + bsum)


_tc_forward = pl.pallas_call(
    _tc_body,
    out_shape=jax.ShapeDtypeStruct((_B, 1), jnp.float32),
)


def kernel(inputs, emb_tables, bias_tables, nas_logits, W1, b1, W2, b2, W3, b3):
    inp = inputs.astype(jnp.int32)
    offs = (jnp.arange(_F, dtype=jnp.int32) * _V)[None, :]
    idx = (inp + offs).reshape(-1)
    bridx = idx >> 3
    fld = (jnp.arange(_F, dtype=jnp.int32) * (_E * _V // 8))[None, :]
    ridx0 = (fld + (inp >> 3)).reshape(-1)
    lane = (inp & 7).reshape(-1)
    lane_exp = jnp.repeat(inp & 7, 8, axis=1)
    emb8 = jnp.transpose(emb_tables, (0, 2, 1)).reshape(_F * _E * _V // 8, 8)
    bias8 = bias_tables.reshape(_F * _V // 8, 8)
    embs, bias8g = _make_sc_gather()(emb8, bias8, ridx0, lane, bridx)
    x = embs.reshape(_B, _F * _E)
    bias8b = bias8g.reshape(_B, _F * 8)
    p = jax.nn.softmax(nas_logits, axis=1).reshape(1, _F * _K)
    out = _tc_forward(x, p, jnp.asarray(_MASKS_BD), bias8b, lane_exp,
                      W1, b1.reshape(1, -1), W2, b2.reshape(1, -1),
                      W3, b3.reshape(1, 1))
    return out.reshape(_B)


# speedup vs baseline: 2.5963x; 1.2317x over previous
import functools

import jax
import jax.numpy as jnp
import numpy as np
from jax import lax
from jax.experimental import pallas as pl
from jax.experimental.pallas import tpu as pltpu
from jax.experimental.pallas import tpu_sc as plsc

_SIZES = [1, 2, 4, 9]
_F = 26
_V = 100000
_E = 16
_B = 4096
_K = len(_SIZES)
_NC, _NS = 2, 16
_NW = _NC * _NS
_N = _B * _F
_RPW = _N // _NW
_CHK = 208


def _nas_masks():
    rows = []
    for i, m in enumerate(_SIZES):
        before = sum(_SIZES[:i])
        rows.append([0.0] * before + [1.0] * m + [0.0] * (_E - m - before))
    return np.array(rows, dtype=np.float32)


def _blockdiag_masks():
    m = _nas_masks()
    out = np.zeros((_F * _K, _F * _E), dtype=np.float32)
    for f in range(_F):
        out[f * _K:(f + 1) * _K, f * _E:(f + 1) * _E] = m
    return out


_MASKS_BD = _blockdiag_masks()

@functools.cache
def _make_sc_bias_gather():
    mesh = plsc.VectorSubcoreMesh(core_axis_name="c", subcore_axis_name="s")

    @functools.partial(
        pl.kernel,
        mesh=mesh,
        compiler_params=pltpu.CompilerParams(
            use_tc_tiling_on_sc=False, needs_layout_passes=False),
        out_type=jax.ShapeDtypeStruct((_N, 8), jnp.float32),
        scratch_types=[
            pltpu.VMEM((_RPW,), jnp.int32),
            pltpu.VMEM((_RPW, 8), jnp.float32),
            pltpu.SemaphoreType.DMA,
        ],
    )
    def _sc_bias(bias8_hbm, bridx_hbm, out_bias8, bridx_v, b8_v, sem_b):
        wid = lax.axis_index("s") * _NC + lax.axis_index("c")
        base = wid * _RPW
        pltpu.sync_copy(bridx_hbm.at[pl.ds(base, _RPW)], bridx_v)
        pltpu.async_copy(bias8_hbm.at[bridx_v], b8_v, sem_b).wait()
        pltpu.sync_copy(b8_v, out_bias8.at[pl.ds(base, _RPW)])

    return _sc_bias


@functools.cache
def _make_sc_gather():
    mesh = plsc.VectorSubcoreMesh(core_axis_name="c", subcore_axis_name="s")

    @functools.partial(
        pl.kernel,
        mesh=mesh,
        compiler_params=pltpu.CompilerParams(
            use_tc_tiling_on_sc=False, needs_layout_passes=False),
        out_type=jax.ShapeDtypeStruct((_N, _E), jnp.float32),
        scratch_types=[
            pltpu.VMEM((_RPW,), jnp.int32),
            pltpu.VMEM((_RPW,), jnp.int32),
            pltpu.VMEM((_E * _CHK,), jnp.int32),
            pltpu.VMEM((_E * _CHK, 8), jnp.float32),
            pltpu.VMEM((_RPW, _E), jnp.float32),
            pltpu.SemaphoreType.DMA,
        ],
    )
    def _sc_gather(emb8_hbm, ridx0_hbm, lane_hbm,
                   out_emb, ridx0_v, lane_v, ge_v, blk_v, emb_v, sem_e):
        wid = lax.axis_index("s") * _NC + lax.axis_index("c")
        base = wid * _RPW
        pltpu.sync_copy(ridx0_hbm.at[pl.ds(base, _RPW)], ridx0_v)
        pltpu.sync_copy(lane_hbm.at[pl.ds(base, _RPW)], lane_v)

        def _chunk(c, _):
            c0 = c * _CHK

            def _fill(g, _):
                j0 = g * 16
                rbase = ridx0_v[pl.ds(c0 + j0, 16)]
                for e in range(_E):
                    ge_v[pl.ds(e * _CHK + j0, 16)] = rbase + e * (_V // 8)
                return 0

            lax.fori_loop(0, _CHK // 16, _fill, 0)
            copies = [
                pltpu.async_copy(
                    emb8_hbm.at[ge_v.at[pl.ds(e * _CHK, _CHK)]],
                    blk_v.at[pl.ds(e * _CHK, _CHK)], sem_e)
                for e in range(_E)
            ]
            for cp in copies:
                cp.wait()

            def _extract(g, _):
                j0 = g * 16
                rows = lax.iota(jnp.int32, 16)
                lanes = lane_v[pl.ds(c0 + j0, 16)]
                dst = emb_v.at[pl.ds(c0 + j0, 16)]
                for e in range(_E):
                    vals = plsc.load_gather(
                        blk_v.at[pl.ds(e * _CHK + j0, 16)], [rows, lanes])
                    plsc.store_scatter(
                        dst, [rows, jnp.full((16,), e, jnp.int32)], vals)
                return 0

            lax.fori_loop(0, _CHK // 16, _extract, 0)
            return 0

        lax.fori_loop(0, _RPW // _CHK, _chunk, 0)
        pltpu.sync_copy(emb_v, out_emb.at[pl.ds(base, _RPW)])

    return _sc_gather


def _tc_body(x_ref, p_ref, m_ref, bias_ref, lane_ref, w1_ref, b1_ref, w2_ref,
             b2_ref, w3_ref, b3_ref, out_ref):
    x = x_ref[...]
    mean = jnp.mean(x, axis=0, keepdims=True)
    xc = x - mean
    var = jnp.mean(xc * xc, axis=0, keepdims=True)
    cm = jnp.dot(p_ref[...], m_ref[...],
                 preferred_element_type=jnp.float32)
    scale = cm * lax.rsqrt(var + 1e-3)
    xn = xc * scale
    h = jnp.dot(xn, w1_ref[...],
                preferred_element_type=jnp.float32) + b1_ref[...]
    h = jnp.maximum(h, 0.0)
    h = jnp.dot(h, w2_ref[...],
                preferred_element_type=jnp.float32) + b2_ref[...]
    h = jnp.maximum(h, 0.0)
    y = jnp.dot(h, w3_ref[...],
                preferred_element_type=jnp.float32) + b3_ref[...]
    b8 = bias_ref[...]
    lanes = lane_ref[...]
    sub = lax.broadcasted_iota(jnp.int32, b8.shape, 1) % 8
    picked = jnp.where(lanes == sub, b8, 0.0)
    bsum = jnp.sum(picked, axis=1, keepdims=True)
    out_ref[...] = jax.nn.sigmoid(y + bsum)


_tc_forward = pl.pallas_call(
    _tc_body,
    out_shape=jax.ShapeDtypeStruct((_B, 1), jnp.float32),
)


def kernel(inputs, emb_tables, bias_tables, nas_logits, W1, b1, W2, b2, W3, b3):
    inp = inputs.astype(jnp.int32)
    offs = (jnp.arange(_F, dtype=jnp.int32) * _V)[None, :]
    idx = (inp + offs).reshape(-1)
    bridx = idx >> 3
    fld = (jnp.arange(_F, dtype=jnp.int32) * (_E * _V // 8))[None, :]
    ridx0 = (fld + (inp >> 3)).reshape(-1)
    lane = (inp & 7).reshape(-1)
    lane_exp = jnp.repeat(inp & 7, 8, axis=1)
    emb8 = jnp.transpose(emb_tables, (0, 2, 1)).reshape(_F * _E * _V // 8, 8)
    bias8 = bias_tables.reshape(_F * _V // 8, 8)
    embs = _make_sc_gather()(emb8, ridx0, lane)
    bias8g = _make_sc_bias_gather()(bias8, bridx)
    x = embs.reshape(_B, _F * _E)
    bias8b = bias8g.reshape(_B, _F * 8)
    p = jax.nn.softmax(nas_logits, axis=1).reshape(1, _F * _K)
    out = _tc_forward(x, p, jnp.asarray(_MASKS_BD), bias8b, lane_exp,
                      W1, b1.reshape(1, -1), W2, b2.reshape(1, -1),
                      W3, b3.reshape(1, 1))
    return out.reshape(_B)
